# baseline (speedup 1.0000x reference)
import functools

import jax
import jax.numpy as jnp
from jax import lax
from jax.experimental import pallas as pl
from jax.experimental.pallas import tpu as pltpu
from jax.experimental.pallas import tpu_sc as plsc

_B, _L, _D, _V = 4096, 200, 128, 100
_N = _B * _L

_NC, _NS = 2, 16
_NW = _NC * _NS
_PER_W = _N // _NW
_CH = 128
_N_CH = _PER_W // _CH


def _prep_body(table_ref, idx_ref, scaled_ref, mask_ref):
    t = table_ref[...]
    sq = jnp.sum(t * t, axis=1, keepdims=True)
    norm = jnp.sqrt(sq + 1e-12)
    scale = jnp.where(norm > 1.0, 1.0 / (norm + 1e-7), 1.0)
    scaled_ref[...] = t * scale
    mask_ref[...] = idx_ref[...] == 0


def _prep(table, idxs):
    return pl.pallas_call(
        _prep_body,
        out_shape=(
            jax.ShapeDtypeStruct((_V + 1, _D), jnp.float32),
            jax.ShapeDtypeStruct((_B, _L), jnp.bool_),
        ),
    )(table, idxs)


_NBUF = 5
_DLAG = 3


def _sc_gather(table, idx_rows):
    mesh = plsc.VectorSubcoreMesh(core_axis_name="c", subcore_axis_name="s")

    @functools.partial(
        pl.kernel,
        mesh=mesh,
        out_type=jax.ShapeDtypeStruct((_N, _D), jnp.float32),
        scratch_types=[
            pltpu.VMEM((_N_CH, _CH), jnp.int32),
            pltpu.VMEM((_NBUF, _CH, _D), jnp.float32),
            pltpu.VMEM_SHARED((_CH, _D), jnp.float32),
        ]
        + [pltpu.SemaphoreType.DMA] * (2 * _NBUF),
    )
    def k(table_hbm, idx_hbm, out_hbm, idx_v, rows_v, table_sh, *sems):
        gsem = sems[:_NBUF]
        osem = sems[_NBUF:]
        sid = lax.axis_index("s")
        wid = sid * _NC + lax.axis_index("c")
        base = wid * _PER_W

        @pl.when(sid == 0)
        def _():
            pltpu.sync_copy(table_hbm, table_sh.at[pl.ds(0, _V + 1)])

        pltpu.sync_copy(idx_hbm.at[pl.ds(wid * _N_CH, _N_CH)], idx_v)
        plsc.subcore_barrier()

        def issue_gather(s, b):
            pltpu.make_async_copy(
                table_sh.at[idx_v.at[s]], rows_v.at[b], gsem[b]
            ).start()

        def wait_gather(b):
            pltpu.make_async_copy(
                table_sh.at[idx_v.at[0]], rows_v.at[b], gsem[b]
            ).wait()

        def issue_out(s, b):
            pltpu.make_async_copy(
                rows_v.at[b], out_hbm.at[pl.ds(base + s * _CH, _CH)], osem[b]
            ).start()

        def wait_out(b):
            pltpu.make_async_copy(
                rows_v.at[b], out_hbm.at[pl.ds(base, _CH)], osem[b]
            ).wait()

        def slot(i, b, first):
            s = i * _NBUF + b
            if not first:
                wait_out(b)
            issue_gather(s, b)
            if not (first and b < _DLAG):
                b2 = (b - _DLAG) % _NBUF
                wait_gather(b2)
                issue_out(s - _DLAG, b2)

        for b in range(_NBUF):
            slot(0, b, True)

        def body(i, carry):
            for b in range(_NBUF):
                slot(i, b, False)
            return carry

        lax.fori_loop(1, _N_CH // _NBUF, body, 0)

        for j in range(_DLAG):
            s2 = _N_CH - _DLAG + j
            b2 = s2 % _NBUF
            wait_gather(b2)
            issue_out(s2, b2)
        for b in range(_NBUF):
            wait_out(b)

    return k(table, idx_rows)


def kernel(positions_in_patch, output_idxs, table):
    scaled_table, mask = _prep(table, output_idxs)
    flat = _sc_gather(scaled_table, output_idxs.reshape(_N // _CH, _CH))
    emb = flat.reshape(_B, _L, _D)
    return (positions_in_patch, emb, mask)


# speedup vs baseline: 1.0127x; 1.0127x over previous
import functools

import jax
import jax.numpy as jnp
from jax import lax
from jax.experimental import pallas as pl
from jax.experimental.pallas import tpu as pltpu
from jax.experimental.pallas import tpu_sc as plsc

_B, _L, _D, _V = 4096, 200, 128, 100
_N = _B * _L

_NC, _NS = 2, 16
_NW = _NC * _NS
_PER_W = _N // _NW
_CH = 128
_N_CH = _PER_W // _CH


def _prep_body(table_ref, idx_ref, scaled_ref, mask_ref):
    t = table_ref[...]
    sq = jnp.sum(t * t, axis=1, keepdims=True)
    norm = jnp.sqrt(sq + 1e-12)
    scale = jnp.where(norm > 1.0, 1.0 / (norm + 1e-7), 1.0)
    scaled_ref[...] = t * scale
    mask_ref[...] = idx_ref[...] == 0


def _prep(table, idxs):
    return pl.pallas_call(
        _prep_body,
        out_shape=(
            jax.ShapeDtypeStruct((_V + 1, _D), jnp.float32),
            jax.ShapeDtypeStruct((_B, _L), jnp.bool_),
        ),
    )(table, idxs)


_ND = 3
_NR = 2
_N_DMA = 120
_REG_POS = (0, 1, 3, 4)
_N_REG = _N_CH - _N_DMA


def _sc_gather(table, idx_rows):
    mesh = plsc.VectorSubcoreMesh(core_axis_name="c", subcore_axis_name="s")

    @functools.partial(
        pl.kernel,
        mesh=mesh,
        out_type=jax.ShapeDtypeStruct((_N, _D), jnp.float32),
        compiler_params=pltpu.CompilerParams(needs_layout_passes=False),
        scratch_types=[
            pltpu.VMEM((_N_CH, _CH), jnp.int32),
            pltpu.VMEM((_V + 1, _D), jnp.float32),
            pltpu.VMEM((_ND, _CH, _D), jnp.float32),
            pltpu.VMEM((_NR, _CH, _D), jnp.float32),
            pltpu.VMEM((_CH * 17,), jnp.int32),
            pltpu.VMEM_SHARED((_V + 1, _D), jnp.float32),
        ]
        + [pltpu.SemaphoreType.DMA] * (2 * _ND + _NR),
    )
    def k(table_hbm, idx_hbm, out_hbm, idx_v, tab_v, rows_d,
          rows_r, idx_bc, table_sh, *sems):
        gsem = sems[:_ND]
        odsem = sems[_ND:2 * _ND]
        orsem = sems[2 * _ND:]
        sid = lax.axis_index("s")
        wid = sid * _NC + lax.axis_index("c")
        base = wid * _PER_W

        @pl.when(sid == 0)
        def _():
            pltpu.sync_copy(table_hbm, table_sh)

        pltpu.sync_copy(table_hbm, tab_v)
        pltpu.sync_copy(idx_hbm.at[pl.ds(wid * _N_CH, _N_CH)], idx_v)
        plsc.subcore_barrier()

        def issue_gather(s, b):
            pltpu.make_async_copy(
                table_sh.at[idx_v.at[s]], rows_d.at[b], gsem[b]
            ).start()

        def wait_gather(b):
            pltpu.make_async_copy(
                table_sh.at[idx_v.at[0]], rows_d.at[b], gsem[b]
            ).wait()

        def issue_out_d(s, b):
            pltpu.make_async_copy(
                rows_d.at[b], out_hbm.at[pl.ds(base + s * _CH, _CH)], odsem[b]
            ).start()

        def wait_out_d(b):
            pltpu.make_async_copy(
                rows_d.at[b], out_hbm.at[pl.ds(base, _CH)], odsem[b]
            ).wait()

        def issue_out_r(s, rb):
            pltpu.make_async_copy(
                rows_r.at[rb], out_hbm.at[pl.ds(base + s * _CH, _CH)], orsem[rb]
            ).start()

        def wait_out_r(rb):
            pltpu.make_async_copy(
                rows_r.at[rb], out_hbm.at[pl.ds(base, _CH)], orsem[rb]
            ).wait()

        lane16 = lax.broadcasted_iota(jnp.int32, (16,), 0)

        lane17 = lane16 * 17

        def build_reg_chunk(s, rb):
            for g in range(_CH // 16):
                seg = idx_v[s, pl.ds(g * 16, 16)]
                for c in range(16):
                    plsc.store_scatter(idx_bc, [lane17 + (g * 272 + c)], seg)

            @plsc.parallel_loop(0, _CH, 1, unroll=4)
            def _(r):
                row = idx_bc[pl.ds(r * 17, 16)]
                for j in range(_D // 16):
                    v = plsc.load_gather(tab_v, [row, lane16 + (j * 16)])
                    rows_r[rb, r, pl.ds(j * 16, 16)] = v

        def slot(i, e, first):
            kk = i * 6 + e
            b = e % 3
            if not (first and kk < _ND):
                wait_out_d(b)
            issue_gather(kk, b)
            if not (first and kk < 1):
                b2 = (e - 1) % 3
                wait_gather(b2)
                issue_out_d(kk - 1, b2)
            if e in _REG_POS:
                m = _REG_POS.index(e)
                rb = m % 2
                if not (first and m < _NR):
                    wait_out_r(rb)
                rc = _N_DMA + i * len(_REG_POS) + m
                build_reg_chunk(rc, rb)
                issue_out_r(rc, rb)

        for e in range(6):
            slot(0, e, True)

        def body(i, carry):
            for e in range(6):
                slot(i, e, False)
            return carry

        lax.fori_loop(1, _N_DMA // 6, body, 0)

        wait_gather((_N_DMA - 1) % 3)
        issue_out_d(_N_DMA - 1, (_N_DMA - 1) % 3)
        for b in range(_ND):
            wait_out_d(b)
        for rb in range(_NR):
            wait_out_r(rb)

    return k(table, idx_rows)


def kernel(positions_in_patch, output_idxs, table):
    scaled_table, mask = _prep(table, output_idxs)
    flat = _sc_gather(scaled_table, output_idxs.reshape(_N // _CH, _CH))
    emb = flat.reshape(_B, _L, _D)
    return (positions_in_patch, emb, mask)
